# Initial kernel scaffold; baseline (speedup 1.0000x reference)
import jax
import jax.numpy as jnp
from jax.experimental import pallas as pl


def kernel(snapshot_readouts, author_h, accum_citations, final_boxes, edge_index, accum_table, W_enc, b_enc, W_out, b_out, W_mix, b_mix):
    raise NotImplementedError("write your pallas kernel here")


# speedup vs baseline: 12.1809x; 12.1809x over previous
import functools

import jax
import jax.numpy as jnp
from jax import lax
from jax.experimental import pallas as pl
from jax.experimental.pallas import tpu as pltpu
from jax.experimental.pallas import tpu_sc as plsc

N_NODES = 10000
N_EDGES = 320000
D = 128
TABLE_PAD = 8

NC = 2
NS = 16
L = 16

N_PAD = 10240
E_PER_TILE = 10240
E_PAD = NC * NS * E_PER_TILE
CHUNK = 128
N_CHUNKS = E_PER_TILE // CHUNK
ROWS_PER_TILE = N_PAD // NS


def _sc_segment_accum():
    mesh = plsc.VectorSubcoreMesh(
        core_axis_name="c", subcore_axis_name="s", num_cores=NC, num_subcores=NS
    )

    @functools.partial(
        pl.kernel,
        out_type=(
            jax.ShapeDtypeStruct((NC, N_NODES, D), jnp.float32),
            jax.ShapeDtypeStruct((NC * N_PAD,), jnp.float32),
        ),
        mesh=mesh,
        scratch_types=[
            pltpu.VMEM((N_NODES,), jnp.float32),
            pltpu.VMEM((CHUNK,), jnp.int32),
            pltpu.VMEM((CHUNK,), jnp.int32),
            pltpu.VMEM((CHUNK,), jnp.float32),
            pltpu.VMEM((CHUNK, D), jnp.float32),
            pltpu.VMEM_SHARED((N_PAD, D), jnp.float32),
            pltpu.VMEM_SHARED((N_PAD,), jnp.float32),
            pltpu.SemaphoreType.DMA,
        ],
        compiler_params=pltpu.CompilerParams(needs_layout_passes=False),
    )
    def seg_kernel(src_hbm, dst_hbm, accum_hbm, author_hbm, s_out, w_out,
                   ec_tab, src_v, dst_v, ec_v, rows_v, s_sh, w_sh, sem):
        cid = lax.axis_index("c")
        sid = lax.axis_index("s")
        zero16 = jnp.zeros((L,), jnp.float32)
        izero16 = jnp.zeros((L,), jnp.int32)

        pltpu.sync_copy(accum_hbm, ec_tab)

        def exp_body(i, _):
            ec_tab[pl.ds(i * L, L)] = jnp.exp(ec_tab[pl.ds(i * L, L)])
            return 0

        lax.fori_loop(0, N_NODES // L, exp_body, 0)

        def zrow_body(b, _):
            for j in range(D // L):
                rows_v[b, pl.ds(j * L, L)] = zero16
            return 0

        lax.fori_loop(0, CHUNK, zrow_body, 0)

        def zw_body(g, _):
            ec_v[pl.ds(g * L, L)] = zero16
            return 0

        lax.fori_loop(0, CHUNK // L, zw_body, 0)

        for t in range(ROWS_PER_TILE // CHUNK):
            pltpu.sync_copy(rows_v, s_sh.at[pl.ds(sid * ROWS_PER_TILE + t * CHUNK, CHUNK)])
            pltpu.sync_copy(ec_v, w_sh.at[pl.ds(sid * ROWS_PER_TILE + t * CHUNK, CHUNK)])
        plsc.subcore_barrier()

        base0 = (cid * NS + sid) * E_PER_TILE

        def chunk_body(k, _):
            base = base0 + k * CHUNK
            pltpu.sync_copy(src_hbm.at[pl.ds(base, CHUNK)], src_v)
            pltpu.sync_copy(dst_hbm.at[pl.ds(base, CHUNK)], dst_v)
            pltpu.async_copy(author_hbm.at[src_v], rows_v, sem).wait()

            def ec_body(g, _):
                idx = src_v[pl.ds(g * L, L)]
                ec_v[pl.ds(g * L, L)] = plsc.load_gather(ec_tab, [idx])
                return 0

            lax.fori_loop(0, CHUNK // L, ec_body, 0)
            pltpu.sync_copy(ec_v, w_sh.at[dst_v], add=True)

            def scale_body(b, _):
                splat = plsc.load_gather(ec_v, [izero16 + b])
                for j in range(D // L):
                    rows_v[b, pl.ds(j * L, L)] = rows_v[b, pl.ds(j * L, L)] * splat
                return 0

            lax.fori_loop(0, CHUNK, scale_body, 0)
            pltpu.sync_copy(rows_v, s_sh.at[dst_v], add=True)
            return 0

        lax.fori_loop(0, N_CHUNKS, chunk_body, 0)
        plsc.subcore_barrier()

        row0 = sid * ROWS_PER_TILE
        full = N_NODES - (NS - 1) * ROWS_PER_TILE

        pltpu.sync_copy(w_sh.at[pl.ds(row0, ROWS_PER_TILE)],
                        w_out.at[pl.ds(cid * N_PAD + row0, ROWS_PER_TILE)])

        @pl.when(sid < NS - 1)
        def _():
            pltpu.sync_copy(s_sh.at[pl.ds(row0, ROWS_PER_TILE)],
                            s_out.at[cid, pl.ds(row0, ROWS_PER_TILE)])

        @pl.when(sid == NS - 1)
        def _():
            pltpu.sync_copy(s_sh.at[pl.ds(row0, full)],
                            s_out.at[cid, pl.ds(row0, full)])

    return seg_kernel


_BLK = 1000
_GRID = N_NODES // _BLK


def _tc_body(s0, s1, w0, w1, snap, boxes, table, w_enc, b_enc, w_out_p, b_out,
             w_mix, b_mix, out_ref, pope_ref, loss_ref):
    i = pl.program_id(0)
    wsum = w0[...] + w1[...]
    paper_h = (s0[...] + s1[...]) / (wsum + 1e-20)

    pe = jnp.dot(snap[...], w_enc[...], preferred_element_type=jnp.float32) + b_enc[...]
    pop_embs = jnp.where(pe > 0, pe, 0.01 * pe)
    pope_ref[...] = pop_embs
    out_ref[...] = (
        jnp.dot(pop_embs, w_out_p[...], preferred_element_type=jnp.float32) + b_out[...]
    )

    wm = w_mix[...]
    tmix = jnp.dot(table[...], wm[D:, :], preferred_element_type=jnp.float32)
    onehot = (boxes[...] == lax.broadcasted_iota(jnp.int32, (1, TABLE_PAD), 1)
              ).astype(jnp.float32)
    mx = (jnp.dot(paper_h, wm[:D, :], preferred_element_type=jnp.float32)
          + jnp.dot(onehot, tmix, preferred_element_type=jnp.float32) + b_mix[...])
    pop_mix = jnp.where(mx > 0, mx, 0.01 * mx)

    n1 = jnp.maximum(jnp.sqrt(jnp.sum(pop_embs * pop_embs, axis=1, keepdims=True)), 1e-8)
    n2 = jnp.maximum(jnp.sqrt(jnp.sum(pop_mix * pop_mix, axis=1, keepdims=True)), 1e-8)
    cos = jnp.sum(pop_embs * pop_mix, axis=1, keepdims=True) / (n1 * n2)
    part = jnp.sum(1.0 - cos) * (1.0 / N_NODES)

    @pl.when(i == 0)
    def _():
        loss_ref[0, 0] = 0.0

    loss_ref[0, 0] += part


def _tc_heads(s0, s1, w0, w1, snap, boxes, table, w_enc, b_enc, w_out_p, b_out,
              w_mix, b_mix):
    full = lambda s: pl.BlockSpec(s, lambda i: (0, 0))
    row = lambda c: pl.BlockSpec((_BLK, c), lambda i: (i, 0))
    return pl.pallas_call(
        _tc_body,
        grid=(_GRID,),
        in_specs=[
            row(D), row(D), row(1), row(1), row(D), row(1),
            full((TABLE_PAD, D)), full((D, D)), full((1, D)), full((D, 1)),
            full((1, 1)), full((2 * D, D)), full((1, D)),
        ],
        out_specs=[
            row(1), row(D),
            pl.BlockSpec((1, 1), lambda i: (0, 0), memory_space=pltpu.SMEM),
        ],
        out_shape=[
            jax.ShapeDtypeStruct((N_NODES, 1), jnp.float32),
            jax.ShapeDtypeStruct((N_NODES, D), jnp.float32),
            jax.ShapeDtypeStruct((1, 1), jnp.float32),
        ],
        compiler_params=pltpu.CompilerParams(
            dimension_semantics=("arbitrary",),
        ),
    )(s0, s1, w0, w1, snap, boxes, table, w_enc, b_enc, w_out_p, b_out,
      w_mix, b_mix)


@jax.jit
def kernel(snapshot_readouts, author_h, accum_citations, final_boxes, edge_index,
           accum_table, W_enc, b_enc, W_out, b_out, W_mix, b_mix):
    src = edge_index[0].astype(jnp.int32)
    dst = edge_index[1].astype(jnp.int32)
    npad = E_PAD - N_EDGES
    src_p = jnp.concatenate([src, jnp.zeros((npad,), jnp.int32)])
    dst_p = jnp.concatenate([dst, jnp.full((npad,), N_NODES, jnp.int32)])

    s_part, w_flat = _sc_segment_accum()(src_p, dst_p, accum_citations, author_h)
    w_part = w_flat.reshape(NC, N_PAD)

    table8 = jnp.zeros((TABLE_PAD, D), jnp.float32).at[:6].set(accum_table)
    out, pop_embs, loss = _tc_heads(
        s_part[0], s_part[1],
        w_part[0, :N_NODES, None], w_part[1, :N_NODES, None],
        snapshot_readouts, final_boxes.astype(jnp.int32)[:, None],
        table8, W_enc, b_enc[None, :], W_out, b_out[None, :],
        W_mix, b_mix[None, :],
    )
    return out, pop_embs, loss[0, 0]


# speedup vs baseline: 13.6455x; 1.1202x over previous
import functools

import jax
import jax.numpy as jnp
from jax import lax
from jax.experimental import pallas as pl
from jax.experimental.pallas import tpu as pltpu
from jax.experimental.pallas import tpu_sc as plsc

N_NODES = 10000
N_EDGES = 320000
D = 128
TABLE_PAD = 8

NC = 2
NS = 16
L = 16

N_PAD = 10240
S_ROWS = 10240
E_PER_TILE = 10240
E_PAD = NC * NS * E_PER_TILE
CHUNK = 80
N_CHUNKS = E_PER_TILE // CHUNK
NBUF = 4
W_PER_TILE = N_PAD // NS
SZ_PER_TILE = S_ROWS // NS
SO_PER_TILE = 624


def _sc_segment_accum():
    mesh = plsc.VectorSubcoreMesh(
        core_axis_name="c", subcore_axis_name="s", num_cores=NC, num_subcores=NS
    )

    @functools.partial(
        pl.kernel,
        out_type=(
            jax.ShapeDtypeStruct((NC, N_NODES, D), jnp.float32),
            jax.ShapeDtypeStruct((NC * N_PAD,), jnp.float32),
        ),
        mesh=mesh,
        scratch_types=[
            pltpu.VMEM((NBUF, CHUNK), jnp.int32),
            pltpu.VMEM((NBUF, CHUNK), jnp.int32),
            pltpu.VMEM((NBUF, CHUNK), jnp.float32),
            pltpu.VMEM((NBUF, CHUNK, D), jnp.float32),
            pltpu.VMEM_SHARED((S_ROWS, D), jnp.float32),
            pltpu.VMEM_SHARED((N_PAD,), jnp.float32),
            [pltpu.SemaphoreType.DMA] * NBUF,
            [pltpu.SemaphoreType.DMA] * NBUF,
            [pltpu.SemaphoreType.DMA] * NBUF,
            [pltpu.SemaphoreType.DMA] * NBUF,
        ],
        compiler_params=pltpu.CompilerParams(needs_layout_passes=False),
    )
    def seg_kernel(src_hbm, dst_hbm, ec_hbm, scaled_hbm, s_out, w_out,
                   src_v, dst_v, ec_v, rows_v, s_sh, w_sh,
                   sem_g, sem_e, sem_s, sem_w):
        cid = lax.axis_index("c")
        sid = lax.axis_index("s")
        zero16 = jnp.zeros((L,), jnp.float32)

        def zrow_body(b, _):
            for j in range(D // L):
                rows_v[0, b, pl.ds(j * L, L)] = zero16
            return 0

        lax.fori_loop(0, CHUNK, zrow_body, 0)

        def zw_body(g, _):
            ec_v[0, pl.ds(g * L, L)] = zero16
            return 0

        lax.fori_loop(0, CHUNK // L, zw_body, 0)

        zrow0 = sid * SZ_PER_TILE
        nfull = SZ_PER_TILE // CHUNK
        for t in range(nfull):
            pltpu.sync_copy(rows_v.at[0], s_sh.at[pl.ds(zrow0 + t * CHUNK, CHUNK)])
        rem = SZ_PER_TILE - nfull * CHUNK
        if rem:
            pltpu.sync_copy(rows_v.at[0, pl.ds(0, rem)],
                            s_sh.at[pl.ds(zrow0 + nfull * CHUNK, rem)])
        for t in range(W_PER_TILE // CHUNK):
            pltpu.sync_copy(ec_v.at[0],
                            w_sh.at[pl.ds(sid * W_PER_TILE + t * CHUNK, CHUNK)])
        plsc.subcore_barrier()

        base0 = (cid * NS + sid) * E_PER_TILE

        def issue_gather(k, b):
            pltpu.sync_copy(src_hbm.at[pl.ds(base0 + k * CHUNK, CHUNK)],
                            src_v.at[b])
            pltpu.sync_copy(dst_hbm.at[pl.ds(base0 + k * CHUNK, CHUNK)],
                            dst_v.at[b])
            pltpu.async_copy(scaled_hbm.at[src_v.at[b]], rows_v.at[b], sem_g[b])
            pltpu.async_copy(ec_hbm.at[src_v.at[b]], ec_v.at[b], sem_e[b])

        def wait_gather(b):
            pltpu.make_async_copy(scaled_hbm.at[src_v.at[b]], rows_v.at[b],
                                  sem_g[b]).wait()
            pltpu.make_async_copy(ec_hbm.at[src_v.at[b]], ec_v.at[b],
                                  sem_e[b]).wait()

        def wait_scatters(b):
            pltpu.make_async_copy(rows_v.at[b], s_sh.at[dst_v.at[b]],
                                  sem_s[b]).wait()
            pltpu.make_async_copy(ec_v.at[b], w_sh.at[dst_v.at[b]],
                                  sem_w[b]).wait()

        for b in range(2):
            issue_gather(b, b)

        def outer_body(kk, _):
            for b in range(NBUF):
                k = kk * NBUF + b
                wait_gather(b)
                pltpu.async_copy(rows_v.at[b], s_sh.at[dst_v.at[b]], sem_s[b],
                                 add=True)
                pltpu.async_copy(ec_v.at[b], w_sh.at[dst_v.at[b]], sem_w[b],
                                 add=True)
                b2 = (b + 2) % NBUF
                k2 = k + 2

                @pl.when(k2 < N_CHUNKS)
                def _():
                    @pl.when(k2 >= NBUF)
                    def _():
                        wait_scatters(b2)

                    issue_gather(k2, b2)

            return 0

        lax.fori_loop(0, N_CHUNKS // NBUF, outer_body, 0)
        for b in range(NBUF):
            wait_scatters(b)
        plsc.subcore_barrier()

        row0 = sid * SO_PER_TILE
        last = N_NODES - (NS - 1) * SO_PER_TILE

        @pl.when(sid < NS - 1)
        def _():
            pltpu.sync_copy(s_sh.at[pl.ds(row0, SO_PER_TILE)],
                            s_out.at[cid, pl.ds(row0, SO_PER_TILE)])

        @pl.when(sid == NS - 1)
        def _():
            pltpu.sync_copy(s_sh.at[pl.ds(row0, last)],
                            s_out.at[cid, pl.ds(row0, last)])
        wrow0 = sid * W_PER_TILE
        pltpu.sync_copy(w_sh.at[pl.ds(wrow0, W_PER_TILE)],
                        w_out.at[pl.ds(cid * N_PAD + wrow0, W_PER_TILE)])

    return seg_kernel


_BLK = 1000
_GRID = N_NODES // _BLK


def _prescale_body(accum, author, ec_ref, scaled_ref):
    ec = jnp.exp(accum[...])
    ec_ref[...] = ec
    scaled_ref[...] = ec * author[...]


def _tc_prescale(accum2d, author_h):
    row = lambda c: pl.BlockSpec((_BLK, c), lambda i: (i, 0))
    return pl.pallas_call(
        _prescale_body,
        grid=(_GRID,),
        in_specs=[row(1), row(D)],
        out_specs=[row(1), row(D)],
        out_shape=[
            jax.ShapeDtypeStruct((N_NODES, 1), jnp.float32),
            jax.ShapeDtypeStruct((N_NODES, D), jnp.float32),
        ],
    )(accum2d, author_h)


def _tc_body(s0, s1, w0, w1, snap, boxes, table, w_enc, b_enc, w_out_p, b_out,
             w_mix, b_mix, out_ref, pope_ref, loss_ref):
    i = pl.program_id(0)
    wsum = w0[...] + w1[...]
    paper_h = (s0[...] + s1[...]) / (wsum + 1e-20)

    pe = jnp.dot(snap[...], w_enc[...], preferred_element_type=jnp.float32) + b_enc[...]
    pop_embs = jnp.where(pe > 0, pe, 0.01 * pe)
    pope_ref[...] = pop_embs
    out_ref[...] = (
        jnp.dot(pop_embs, w_out_p[...], preferred_element_type=jnp.float32) + b_out[...]
    )

    wm = w_mix[...]
    tmix = jnp.dot(table[...], wm[D:, :], preferred_element_type=jnp.float32)
    onehot = (boxes[...] == lax.broadcasted_iota(jnp.int32, (1, TABLE_PAD), 1)
              ).astype(jnp.float32)
    mx = (jnp.dot(paper_h, wm[:D, :], preferred_element_type=jnp.float32)
          + jnp.dot(onehot, tmix, preferred_element_type=jnp.float32) + b_mix[...])
    pop_mix = jnp.where(mx > 0, mx, 0.01 * mx)

    n1 = jnp.maximum(jnp.sqrt(jnp.sum(pop_embs * pop_embs, axis=1, keepdims=True)), 1e-8)
    n2 = jnp.maximum(jnp.sqrt(jnp.sum(pop_mix * pop_mix, axis=1, keepdims=True)), 1e-8)
    cos = jnp.sum(pop_embs * pop_mix, axis=1, keepdims=True) / (n1 * n2)
    part = jnp.sum(1.0 - cos) * (1.0 / N_NODES)

    @pl.when(i == 0)
    def _():
        loss_ref[0, 0] = 0.0

    loss_ref[0, 0] += part


def _tc_heads(s0, s1, w0, w1, snap, boxes, table, w_enc, b_enc, w_out_p, b_out,
              w_mix, b_mix):
    full = lambda s: pl.BlockSpec(s, lambda i: (0, 0))
    row = lambda c: pl.BlockSpec((_BLK, c), lambda i: (i, 0))
    return pl.pallas_call(
        _tc_body,
        grid=(_GRID,),
        in_specs=[
            row(D), row(D), row(1), row(1), row(D), row(1),
            full((TABLE_PAD, D)), full((D, D)), full((1, D)), full((D, 1)),
            full((1, 1)), full((2 * D, D)), full((1, D)),
        ],
        out_specs=[
            row(1), row(D),
            pl.BlockSpec((1, 1), lambda i: (0, 0), memory_space=pltpu.SMEM),
        ],
        out_shape=[
            jax.ShapeDtypeStruct((N_NODES, 1), jnp.float32),
            jax.ShapeDtypeStruct((N_NODES, D), jnp.float32),
            jax.ShapeDtypeStruct((1, 1), jnp.float32),
        ],
        compiler_params=pltpu.CompilerParams(
            dimension_semantics=("arbitrary",),
        ),
    )(s0, s1, w0, w1, snap, boxes, table, w_enc, b_enc, w_out_p, b_out,
      w_mix, b_mix)


@jax.jit
def kernel(snapshot_readouts, author_h, accum_citations, final_boxes, edge_index,
           accum_table, W_enc, b_enc, W_out, b_out, W_mix, b_mix):
    src = edge_index[0].astype(jnp.int32)
    dst = edge_index[1].astype(jnp.int32)
    npad = E_PAD - N_EDGES
    src_p = jnp.concatenate([src, jnp.zeros((npad,), jnp.int32)])
    dst_p = jnp.concatenate([dst, jnp.full((npad,), N_NODES, jnp.int32)])

    ec2d, scaled_h = _tc_prescale(accum_citations[:, None], author_h)
    s_part, w_flat = _sc_segment_accum()(
        src_p, dst_p, ec2d.reshape(N_NODES), scaled_h)
    w_part = w_flat.reshape(NC, N_PAD)

    table8 = jnp.zeros((TABLE_PAD, D), jnp.float32).at[:6].set(accum_table)
    out, pop_embs, loss = _tc_heads(
        s_part[0], s_part[1],
        w_part[0, :N_NODES, None], w_part[1, :N_NODES, None],
        snapshot_readouts, final_boxes.astype(jnp.int32)[:, None],
        table8, W_enc, b_enc[None, :], W_out, b_out[None, :],
        W_mix, b_mix[None, :],
    )
    return out, pop_embs, loss[0, 0]


# speedup vs baseline: 16.9701x; 1.2436x over previous
import functools

import jax
import jax.numpy as jnp
from jax import lax
from jax.experimental import pallas as pl
from jax.experimental.pallas import tpu as pltpu
from jax.experimental.pallas import tpu_sc as plsc

N_NODES = 10000
N_EDGES = 320000
D = 128
TABLE_PAD = 8

NC = 2
NS = 16
L = 16

N_PAD = 10240
S_ROWS = 10240
E_PER_TILE = 10240
E_PAD = NC * NS * E_PER_TILE
CHUNK = 80
N_CHUNKS = E_PER_TILE // CHUNK
NBUF = 4
IDXG = 8
W_PER_TILE = N_PAD // NS
SZ_PER_TILE = S_ROWS // NS
SO_PER_TILE = 624


def _sc_segment_accum():
    mesh = plsc.VectorSubcoreMesh(
        core_axis_name="c", subcore_axis_name="s", num_cores=NC, num_subcores=NS
    )

    @functools.partial(
        pl.kernel,
        out_type=(
            jax.ShapeDtypeStruct((NC, N_NODES, D), jnp.float32),
            jax.ShapeDtypeStruct((NC * N_PAD,), jnp.float32),
        ),
        mesh=mesh,
        scratch_types=[
            pltpu.VMEM((2, IDXG, CHUNK), jnp.int32),
            pltpu.VMEM((2, IDXG, CHUNK), jnp.int32),
            pltpu.VMEM((NBUF, CHUNK), jnp.float32),
            pltpu.VMEM((NBUF, CHUNK, D), jnp.float32),
            pltpu.VMEM_SHARED((S_ROWS, D), jnp.float32),
            pltpu.VMEM_SHARED((N_PAD,), jnp.float32),
            [pltpu.SemaphoreType.DMA] * NBUF,
            [pltpu.SemaphoreType.DMA] * NBUF,
            [pltpu.SemaphoreType.DMA] * NBUF,
            [pltpu.SemaphoreType.DMA] * NBUF,
        ],
        compiler_params=pltpu.CompilerParams(needs_layout_passes=False),
    )
    def seg_kernel(src_hbm, dst_hbm, ec_hbm, scaled_hbm, s_out, w_out,
                   src_v, dst_v, ec_v, rows_v, s_sh, w_sh,
                   sem_g, sem_e, sem_s, sem_w):
        cid = lax.axis_index("c")
        sid = lax.axis_index("s")
        zero16 = jnp.zeros((L,), jnp.float32)

        def zrow_body(b, _):
            for j in range(D // L):
                rows_v[0, b, pl.ds(j * L, L)] = zero16
            return 0

        lax.fori_loop(0, CHUNK, zrow_body, 0)

        def zw_body(g, _):
            ec_v[0, pl.ds(g * L, L)] = zero16
            return 0

        lax.fori_loop(0, CHUNK // L, zw_body, 0)

        zrow0 = sid * SZ_PER_TILE
        nfull = SZ_PER_TILE // CHUNK
        for t in range(nfull):
            pltpu.sync_copy(rows_v.at[0], s_sh.at[pl.ds(zrow0 + t * CHUNK, CHUNK)])
        rem = SZ_PER_TILE - nfull * CHUNK
        if rem:
            pltpu.sync_copy(rows_v.at[0, pl.ds(0, rem)],
                            s_sh.at[pl.ds(zrow0 + nfull * CHUNK, rem)])
        for t in range(W_PER_TILE // CHUNK):
            pltpu.sync_copy(ec_v.at[0],
                            w_sh.at[pl.ds(sid * W_PER_TILE + t * CHUNK, CHUNK)])
        plsc.subcore_barrier()

        cbase = (cid * NS + sid) * N_CHUNKS

        def load_idx_group(g):
            gb = lax.rem(g, 2)
            pltpu.sync_copy(src_hbm.at[pl.ds(cbase + g * IDXG, IDXG)],
                            src_v.at[gb])
            pltpu.sync_copy(dst_hbm.at[pl.ds(cbase + g * IDXG, IDXG)],
                            dst_v.at[gb])

        def src_row(k):
            return src_v.at[lax.rem(k // IDXG, 2), lax.rem(k, IDXG)]

        def dst_row(k):
            return dst_v.at[lax.rem(k // IDXG, 2), lax.rem(k, IDXG)]

        def issue_gather(k, b):
            pltpu.async_copy(scaled_hbm.at[src_row(k)], rows_v.at[b], sem_g[b])
            pltpu.async_copy(ec_hbm.at[src_row(k)], ec_v.at[b], sem_e[b])

        def wait_gather(k, b):
            pltpu.make_async_copy(scaled_hbm.at[src_row(k)], rows_v.at[b],
                                  sem_g[b]).wait()
            pltpu.make_async_copy(ec_hbm.at[src_row(k)], ec_v.at[b],
                                  sem_e[b]).wait()

        def wait_scatters(k, b):
            pltpu.make_async_copy(rows_v.at[b], s_sh.at[dst_row(k)],
                                  sem_s[b]).wait()
            pltpu.make_async_copy(ec_v.at[b], w_sh.at[dst_row(k)],
                                  sem_w[b]).wait()

        load_idx_group(0)
        for b in range(2):
            issue_gather(b, b)

        def outer_body(kk, _):
            for b in range(NBUF):
                k = kk * NBUF + b
                wait_gather(k, b)
                pltpu.async_copy(rows_v.at[b], s_sh.at[dst_row(k)], sem_s[b],
                                 add=True)
                pltpu.async_copy(ec_v.at[b], w_sh.at[dst_row(k)], sem_w[b],
                                 add=True)
                b2 = (b + 2) % NBUF
                k2 = k + 2

                @pl.when(k2 < N_CHUNKS)
                def _():
                    @pl.when(k2 >= NBUF)
                    def _():
                        wait_scatters(k2 - NBUF, b2)

                    @pl.when(lax.rem(k2, IDXG) == 0)
                    def _():
                        load_idx_group(k2 // IDXG)

                    issue_gather(k2, b2)

            return 0

        lax.fori_loop(0, N_CHUNKS // NBUF, outer_body, 0)
        for b in range(NBUF):
            wait_scatters(N_CHUNKS - NBUF + b, b)
        plsc.subcore_barrier()

        row0 = sid * SO_PER_TILE
        last = N_NODES - (NS - 1) * SO_PER_TILE

        @pl.when(sid < NS - 1)
        def _():
            pltpu.sync_copy(s_sh.at[pl.ds(row0, SO_PER_TILE)],
                            s_out.at[cid, pl.ds(row0, SO_PER_TILE)])

        @pl.when(sid == NS - 1)
        def _():
            pltpu.sync_copy(s_sh.at[pl.ds(row0, last)],
                            s_out.at[cid, pl.ds(row0, last)])
        wrow0 = sid * W_PER_TILE
        pltpu.sync_copy(w_sh.at[pl.ds(wrow0, W_PER_TILE)],
                        w_out.at[pl.ds(cid * N_PAD + wrow0, W_PER_TILE)])

    return seg_kernel


_BLK = 1000
_GRID = N_NODES // _BLK


def _prescale_body(accum, author, ec_ref, scaled_ref):
    ec = jnp.exp(accum[...])
    ec_ref[...] = ec
    scaled_ref[...] = ec * author[...]


def _tc_prescale(accum2d, author_h):
    row = lambda c: pl.BlockSpec((_BLK, c), lambda i: (i, 0))
    return pl.pallas_call(
        _prescale_body,
        grid=(_GRID,),
        in_specs=[row(1), row(D)],
        out_specs=[row(1), row(D)],
        out_shape=[
            jax.ShapeDtypeStruct((N_NODES, 1), jnp.float32),
            jax.ShapeDtypeStruct((N_NODES, D), jnp.float32),
        ],
    )(accum2d, author_h)


def _tc_body(s0, s1, w0, w1, snap, boxes, table, w_enc, b_enc, w_out_p, b_out,
             w_mix, b_mix, out_ref, pope_ref, loss_ref):
    i = pl.program_id(0)
    wsum = w0[...] + w1[...]
    paper_h = (s0[...] + s1[...]) / (wsum + 1e-20)

    pe = jnp.dot(snap[...], w_enc[...], preferred_element_type=jnp.float32) + b_enc[...]
    pop_embs = jnp.where(pe > 0, pe, 0.01 * pe)
    pope_ref[...] = pop_embs
    out_ref[...] = (
        jnp.dot(pop_embs, w_out_p[...], preferred_element_type=jnp.float32) + b_out[...]
    )

    wm = w_mix[...]
    tmix = jnp.dot(table[...], wm[D:, :], preferred_element_type=jnp.float32)
    onehot = (boxes[...] == lax.broadcasted_iota(jnp.int32, (1, TABLE_PAD), 1)
              ).astype(jnp.float32)
    mx = (jnp.dot(paper_h, wm[:D, :], preferred_element_type=jnp.float32)
          + jnp.dot(onehot, tmix, preferred_element_type=jnp.float32) + b_mix[...])
    pop_mix = jnp.where(mx > 0, mx, 0.01 * mx)

    n1 = jnp.maximum(jnp.sqrt(jnp.sum(pop_embs * pop_embs, axis=1, keepdims=True)), 1e-8)
    n2 = jnp.maximum(jnp.sqrt(jnp.sum(pop_mix * pop_mix, axis=1, keepdims=True)), 1e-8)
    cos = jnp.sum(pop_embs * pop_mix, axis=1, keepdims=True) / (n1 * n2)
    part = jnp.sum(1.0 - cos) * (1.0 / N_NODES)

    @pl.when(i == 0)
    def _():
        loss_ref[0, 0] = 0.0

    loss_ref[0, 0] += part


def _tc_heads(s0, s1, w0, w1, snap, boxes, table, w_enc, b_enc, w_out_p, b_out,
              w_mix, b_mix):
    full = lambda s: pl.BlockSpec(s, lambda i: (0, 0))
    row = lambda c: pl.BlockSpec((_BLK, c), lambda i: (i, 0))
    return pl.pallas_call(
        _tc_body,
        grid=(_GRID,),
        in_specs=[
            row(D), row(D), row(1), row(1), row(D), row(1),
            full((TABLE_PAD, D)), full((D, D)), full((1, D)), full((D, 1)),
            full((1, 1)), full((2 * D, D)), full((1, D)),
        ],
        out_specs=[
            row(1), row(D),
            pl.BlockSpec((1, 1), lambda i: (0, 0), memory_space=pltpu.SMEM),
        ],
        out_shape=[
            jax.ShapeDtypeStruct((N_NODES, 1), jnp.float32),
            jax.ShapeDtypeStruct((N_NODES, D), jnp.float32),
            jax.ShapeDtypeStruct((1, 1), jnp.float32),
        ],
        compiler_params=pltpu.CompilerParams(
            dimension_semantics=("arbitrary",),
        ),
    )(s0, s1, w0, w1, snap, boxes, table, w_enc, b_enc, w_out_p, b_out,
      w_mix, b_mix)


@jax.jit
def kernel(snapshot_readouts, author_h, accum_citations, final_boxes, edge_index,
           accum_table, W_enc, b_enc, W_out, b_out, W_mix, b_mix):
    src = edge_index[0].astype(jnp.int32)
    dst = edge_index[1].astype(jnp.int32)
    npad = E_PAD - N_EDGES
    src_p = jnp.concatenate([src, jnp.zeros((npad,), jnp.int32)]
                            ).reshape(E_PAD // CHUNK, CHUNK)
    dst_p = jnp.concatenate([dst, jnp.full((npad,), N_NODES, jnp.int32)]
                            ).reshape(E_PAD // CHUNK, CHUNK)

    ec2d, scaled_h = _tc_prescale(accum_citations[:, None], author_h)
    s_part, w_flat = _sc_segment_accum()(
        src_p, dst_p, ec2d.reshape(N_NODES), scaled_h)
    w_part = w_flat.reshape(NC, N_PAD)

    table8 = jnp.zeros((TABLE_PAD, D), jnp.float32).at[:6].set(accum_table)
    out, pop_embs, loss = _tc_heads(
        s_part[0], s_part[1],
        w_part[0, :N_NODES, None], w_part[1, :N_NODES, None],
        snapshot_readouts, final_boxes.astype(jnp.int32)[:, None],
        table8, W_enc, b_enc[None, :], W_out, b_out[None, :],
        W_mix, b_mix[None, :],
    )
    return out, pop_embs, loss[0, 0]


# speedup vs baseline: 16.9737x; 1.0002x over previous
import functools

import jax
import jax.numpy as jnp
from jax import lax
from jax.experimental import pallas as pl
from jax.experimental.pallas import tpu as pltpu
from jax.experimental.pallas import tpu_sc as plsc

N_NODES = 10000
N_EDGES = 320000
D = 128
TABLE_PAD = 8

NC = 2
NS = 16
L = 16

N_PAD = 10240
S_ROWS = 10240
E_PER_TILE = 10240
E_PAD = NC * NS * E_PER_TILE
CHUNK = 80
N_CHUNKS = E_PER_TILE // CHUNK
NBUF = 4
IDXG = 8
W_PER_TILE = N_PAD // NS
SZ_PER_TILE = S_ROWS // NS
SO_PER_TILE = 624


def _sc_segment_accum():
    mesh = plsc.VectorSubcoreMesh(
        core_axis_name="c", subcore_axis_name="s", num_cores=NC, num_subcores=NS
    )

    @functools.partial(
        pl.kernel,
        out_type=(
            jax.ShapeDtypeStruct((NC, N_NODES, D), jnp.float32),
            jax.ShapeDtypeStruct((NC * N_PAD,), jnp.float32),
        ),
        mesh=mesh,
        scratch_types=[
            pltpu.VMEM((2, IDXG, CHUNK), jnp.int32),
            pltpu.VMEM((2, IDXG, CHUNK), jnp.int32),
            pltpu.VMEM((NBUF, CHUNK), jnp.float32),
            pltpu.VMEM((NBUF, CHUNK, D), jnp.float32),
            pltpu.VMEM_SHARED((S_ROWS, D), jnp.float32),
            pltpu.VMEM_SHARED((N_PAD,), jnp.float32),
            [pltpu.SemaphoreType.DMA] * NBUF,
            [pltpu.SemaphoreType.DMA] * NBUF,
            [pltpu.SemaphoreType.DMA] * NBUF,
            [pltpu.SemaphoreType.DMA] * NBUF,
        ],
        compiler_params=pltpu.CompilerParams(needs_layout_passes=False),
    )
    def seg_kernel(src_hbm, dst_hbm, ec_hbm, scaled_hbm, s_out, w_out,
                   src_v, dst_v, ec_v, rows_v, s_sh, w_sh,
                   sem_g, sem_e, sem_s, sem_w):
        cid = lax.axis_index("c")
        sid = lax.axis_index("s")
        zero16 = jnp.zeros((L,), jnp.float32)

        def zrow_body(b, _):
            for j in range(D // L):
                rows_v[0, b, pl.ds(j * L, L)] = zero16
            return 0

        lax.fori_loop(0, CHUNK, zrow_body, 0)

        def zw_body(g, _):
            ec_v[0, pl.ds(g * L, L)] = zero16
            return 0

        lax.fori_loop(0, CHUNK // L, zw_body, 0)

        zrow0 = sid * SZ_PER_TILE
        nfull = SZ_PER_TILE // CHUNK
        for t in range(nfull):
            pltpu.sync_copy(rows_v.at[0], s_sh.at[pl.ds(zrow0 + t * CHUNK, CHUNK)])
        rem = SZ_PER_TILE - nfull * CHUNK
        if rem:
            pltpu.sync_copy(rows_v.at[0, pl.ds(0, rem)],
                            s_sh.at[pl.ds(zrow0 + nfull * CHUNK, rem)])
        for t in range(W_PER_TILE // CHUNK):
            pltpu.sync_copy(ec_v.at[0],
                            w_sh.at[pl.ds(sid * W_PER_TILE + t * CHUNK, CHUNK)])
        plsc.subcore_barrier()

        cbase = (cid * NS + sid) * N_CHUNKS

        def load_idx_group(g):
            gb = lax.rem(g, 2)
            pltpu.sync_copy(src_hbm.at[pl.ds(cbase + g * IDXG, IDXG)],
                            src_v.at[gb])
            pltpu.sync_copy(dst_hbm.at[pl.ds(cbase + g * IDXG, IDXG)],
                            dst_v.at[gb])

        def src_row(k):
            return src_v.at[lax.rem(k // IDXG, 2), lax.rem(k, IDXG)]

        def dst_row(k):
            return dst_v.at[lax.rem(k // IDXG, 2), lax.rem(k, IDXG)]

        def issue_gather(k, b):
            pltpu.async_copy(scaled_hbm.at[src_row(k)], rows_v.at[b], sem_g[b])
            pltpu.async_copy(ec_hbm.at[src_row(k)], ec_v.at[b], sem_e[b])

        def wait_gather(k, b):
            pltpu.make_async_copy(scaled_hbm.at[src_row(k)], rows_v.at[b],
                                  sem_g[b]).wait()
            pltpu.make_async_copy(ec_hbm.at[src_row(k)], ec_v.at[b],
                                  sem_e[b]).wait()

        def wait_scatters(k, b):
            pltpu.make_async_copy(rows_v.at[b], s_sh.at[dst_row(k)],
                                  sem_s[b]).wait()
            pltpu.make_async_copy(ec_v.at[b], w_sh.at[dst_row(k)],
                                  sem_w[b]).wait()

        load_idx_group(0)
        for b in range(2):
            issue_gather(b, b)

        def outer_body(kk, _):
            for b in range(NBUF):
                k = kk * NBUF + b
                wait_gather(k, b)
                pltpu.async_copy(rows_v.at[b], s_sh.at[dst_row(k)], sem_s[b],
                                 add=True)
                pltpu.async_copy(ec_v.at[b], w_sh.at[dst_row(k)], sem_w[b],
                                 add=True)
                b2 = (b + 2) % NBUF
                k2 = k + 2

                @pl.when(k2 < N_CHUNKS)
                def _():
                    @pl.when(k2 >= NBUF)
                    def _():
                        wait_scatters(k2 - NBUF, b2)

                    @pl.when(lax.rem(k2, IDXG) == 0)
                    def _():
                        load_idx_group(k2 // IDXG)

                    issue_gather(k2, b2)

            return 0

        lax.fori_loop(0, N_CHUNKS // NBUF, outer_body, 0)
        for b in range(NBUF):
            wait_scatters(N_CHUNKS - NBUF + b, b)
        plsc.subcore_barrier()

        row0 = sid * SO_PER_TILE
        last = N_NODES - (NS - 1) * SO_PER_TILE

        @pl.when(sid < NS - 1)
        def _():
            pltpu.sync_copy(s_sh.at[pl.ds(row0, SO_PER_TILE)],
                            s_out.at[cid, pl.ds(row0, SO_PER_TILE)])

        @pl.when(sid == NS - 1)
        def _():
            pltpu.sync_copy(s_sh.at[pl.ds(row0, last)],
                            s_out.at[cid, pl.ds(row0, last)])
        wrow0 = sid * W_PER_TILE
        pltpu.sync_copy(w_sh.at[pl.ds(wrow0, W_PER_TILE)],
                        w_out.at[pl.ds(cid * N_PAD + wrow0, W_PER_TILE)])

    return seg_kernel


_BLK = 1000
_GRID = N_NODES // _BLK


def _prescale_body(accum, author, ec_ref, scaled_ref):
    ec = jnp.exp(accum[...])
    ec_ref[...] = ec
    scaled_ref[...] = ec * author[...]


def _tc_prescale(accum2d, author_h):
    row = lambda c: pl.BlockSpec((_BLK, c), lambda i: (i, 0))
    return pl.pallas_call(
        _prescale_body,
        grid=(_GRID,),
        in_specs=[row(1), row(D)],
        out_specs=[row(1), row(D)],
        out_shape=[
            jax.ShapeDtypeStruct((N_NODES, 1), jnp.float32),
            jax.ShapeDtypeStruct((N_NODES, D), jnp.float32),
        ],
    )(accum2d, author_h)


def _tc_body(s0, s1, w0, w1, snap, boxes, table, w_enc, b_enc, w_out_p, b_out,
             w_mix, b_mix, out_ref, pope_ref, loss_ref):
    i = pl.program_id(0)
    wsum = w0[...] + w1[...]
    paper_h = (s0[...] + s1[...]) / (wsum + 1e-20)

    pe = jnp.dot(snap[...], w_enc[...], preferred_element_type=jnp.float32) + b_enc[...]
    pop_embs = jnp.where(pe > 0, pe, 0.01 * pe)
    pope_ref[...] = pop_embs
    out_ref[...] = (
        jnp.dot(pop_embs, w_out_p[...], preferred_element_type=jnp.float32) + b_out[...]
    )

    wm = w_mix[...]
    tmix = jnp.dot(table[...], wm[D:, :], preferred_element_type=jnp.float32)
    onehot = (boxes[...] == lax.broadcasted_iota(jnp.int32, (1, TABLE_PAD), 1)
              ).astype(jnp.float32)
    mx = (jnp.dot(paper_h, wm[:D, :], preferred_element_type=jnp.float32)
          + jnp.dot(onehot, tmix, preferred_element_type=jnp.float32) + b_mix[...])
    pop_mix = jnp.where(mx > 0, mx, 0.01 * mx)

    n1 = jnp.maximum(jnp.sqrt(jnp.sum(pop_embs * pop_embs, axis=1, keepdims=True)), 1e-8)
    n2 = jnp.maximum(jnp.sqrt(jnp.sum(pop_mix * pop_mix, axis=1, keepdims=True)), 1e-8)
    cos = jnp.sum(pop_embs * pop_mix, axis=1, keepdims=True) / (n1 * n2)
    part = jnp.sum(1.0 - cos) * (1.0 / N_NODES)

    @pl.when(i == 0)
    def _():
        loss_ref[0, 0] = 0.0

    loss_ref[0, 0] += part


def _tc_heads(s0, s1, w0, w1, snap, boxes, table, w_enc, b_enc, w_out_p, b_out,
              w_mix, b_mix):
    full = lambda s: pl.BlockSpec(s, lambda i: (0, 0))
    row = lambda c: pl.BlockSpec((_BLK, c), lambda i: (i, 0))
    return pl.pallas_call(
        _tc_body,
        grid=(_GRID,),
        in_specs=[
            row(D), row(D), row(1), row(1), row(D), row(1),
            full((TABLE_PAD, D)), full((D, D)), full((1, D)), full((D, 1)),
            full((1, 1)), full((2 * D, D)), full((1, D)),
        ],
        out_specs=[
            row(1), row(D),
            pl.BlockSpec((1, 1), lambda i: (0, 0), memory_space=pltpu.SMEM),
        ],
        out_shape=[
            jax.ShapeDtypeStruct((N_NODES, 1), jnp.float32),
            jax.ShapeDtypeStruct((N_NODES, D), jnp.float32),
            jax.ShapeDtypeStruct((1, 1), jnp.float32),
        ],
        compiler_params=pltpu.CompilerParams(
            dimension_semantics=("arbitrary",),
        ),
    )(s0, s1, w0, w1, snap, boxes, table, w_enc, b_enc, w_out_p, b_out,
      w_mix, b_mix)


@jax.jit
def kernel(snapshot_readouts, author_h, accum_citations, final_boxes, edge_index,
           accum_table, W_enc, b_enc, W_out, b_out, W_mix, b_mix):
    src = edge_index[0].astype(jnp.int32)
    dst = edge_index[1].astype(jnp.int32)
    npad = E_PAD - N_EDGES
    src_p = jnp.concatenate([src, jnp.zeros((npad,), jnp.int32)]
                            ).reshape(E_PAD // CHUNK, CHUNK)
    pad_dst = N_NODES + (jnp.arange(npad, dtype=jnp.int32) % (S_ROWS - N_NODES))
    dst_p = jnp.concatenate([dst, pad_dst]).reshape(E_PAD // CHUNK, CHUNK)

    ec2d, scaled_h = _tc_prescale(accum_citations[:, None], author_h)
    s_part, w_flat = _sc_segment_accum()(
        src_p, dst_p, ec2d.reshape(N_NODES), scaled_h)
    w_part = w_flat.reshape(NC, N_PAD)

    table8 = jnp.zeros((TABLE_PAD, D), jnp.float32).at[:6].set(accum_table)
    out, pop_embs, loss = _tc_heads(
        s_part[0], s_part[1],
        w_part[0, :N_NODES, None], w_part[1, :N_NODES, None],
        snapshot_readouts, final_boxes.astype(jnp.int32)[:, None],
        table8, W_enc, b_enc[None, :], W_out, b_out[None, :],
        W_mix, b_mix[None, :],
    )
    return out, pop_embs, loss[0, 0]
